# Initial kernel scaffold; baseline (speedup 1.0000x reference)
import jax
import jax.numpy as jnp
from jax.experimental import pallas as pl


def kernel(f_atoms, f_bonds, b2a, a_scope, Wi_w, Wi_b, Wm_w, Wm_b, Wa_w, Wa_b, node_w, node_b, edge_w, edge_b, g1_w, g1_b, g2_w, g2_b):
    raise NotImplementedError("write your pallas kernel here")


# speedup vs baseline: 1.0121x; 1.0121x over previous
import functools

import jax
import jax.numpy as jnp
from jax import lax
from jax.experimental import pallas as pl
from jax.experimental.pallas import tpu as pltpu
from jax.experimental.pallas import tpu_sc as plsc

NB = 320000
NA = 10000
PW = 128
NP = 3
HP = PW * NP
NM = 400
MOL = 25
AF = 128
BF = 144
EF = 16

_NC, _NS = 2, 16
_NW = _NC * _NS
_SCH = 80
NAP = 10240
_RPS = NAP // _NS

_PREC = jax.lax.Precision.HIGHEST
_f32 = jnp.float32


def _sc_scatter3(ha, hb, hc, idx, zrows):
    mesh = plsc.VectorSubcoreMesh(core_axis_name="c", subcore_axis_name="s")
    out = jax.ShapeDtypeStruct((NAP, PW), _f32)

    @functools.partial(
        pl.kernel,
        out_type=[out, out, out, out],
        mesh=mesh,
        scratch_types=[
            pltpu.VMEM_SHARED((NAP, PW), _f32),
            pltpu.VMEM((_SCH, PW), _f32),
            pltpu.VMEM((_SCH,), jnp.int32),
        ],
    )
    def k(a_hbm, b_hbm, c_hbm, idx_hbm, z_hbm, sa, sb, sc0, sc1,
          acc, buf, idx_v):
        c = lax.axis_index("c")
        s = lax.axis_index("s")
        row0 = s * _RPS

        def scan(x_hbm, bond0, nbonds):
            @pl.loop(0, nbonds, step=_SCH)
            def _(j):
                pltpu.sync_copy(idx_hbm.at[pl.ds(bond0 + j, _SCH)], idx_v)
                pltpu.sync_copy(x_hbm.at[pl.ds(bond0 + j, _SCH)], buf)
                pltpu.sync_copy(buf, acc.at[idx_v], add=True)

        pltpu.sync_copy(z_hbm, acc.at[pl.ds(row0, _RPS)])
        plsc.subcore_barrier()

        @pl.when(c == 0)
        def _():
            scan(a_hbm, s * (NB // _NS), NB // _NS)

        @pl.when(c == 1)
        def _():
            scan(b_hbm, s * (NB // _NS), NB // _NS)

        plsc.subcore_barrier()

        @pl.when(c == 0)
        def _():
            pltpu.sync_copy(acc.at[pl.ds(row0, _RPS)], sa.at[pl.ds(row0, _RPS)])

        @pl.when(c == 1)
        def _():
            pltpu.sync_copy(acc.at[pl.ds(row0, _RPS)], sb.at[pl.ds(row0, _RPS)])

        pltpu.sync_copy(z_hbm, acc.at[pl.ds(row0, _RPS)])
        plsc.subcore_barrier()
        scan(c_hbm, (c * _NS + s) * (NB // _NW), NB // _NW)
        plsc.subcore_barrier()

        @pl.when(c == 0)
        def _():
            pltpu.sync_copy(acc.at[pl.ds(row0, _RPS)], sc0.at[pl.ds(row0, _RPS)])

        @pl.when(c == 1)
        def _():
            pltpu.sync_copy(acc.at[pl.ds(row0, _RPS)], sc1.at[pl.ds(row0, _RPS)])

    return k(ha, hb, hc, idx, zrows)


def _sc_gather(t, idx):
    mesh = plsc.VectorSubcoreMesh(core_axis_name="c", subcore_axis_name="s")

    @functools.partial(
        pl.kernel,
        out_type=jax.ShapeDtypeStruct((NB, HP), _f32),
        mesh=mesh,
        scratch_types=[
            pltpu.VMEM((_SCH, HP), _f32),
            pltpu.VMEM((_SCH,), jnp.int32),
            pltpu.SemaphoreType.DMA,
        ],
    )
    def k(t_hbm, idx_hbm, g_hbm, rows_v, idx_v, sem):
        wid = lax.axis_index("c") * _NS + lax.axis_index("s")
        base = wid * (NB // _NW)

        @pl.loop(0, NB // _NW, step=_SCH)
        def _(j):
            pltpu.sync_copy(idx_hbm.at[pl.ds(base + j, _SCH)], idx_v)
            pltpu.async_copy(t_hbm.at[idx_v], rows_v, sem).wait()
            pltpu.sync_copy(rows_v, g_hbm.at[pl.ds(base + j, _SCH)])

    return k(t, idx)


def _dotT(x, w):
    return lax.dot_general(x, w, (((1,), (1,)), ((), ())),
                           preferred_element_type=_f32, precision=_PREC)


def _split3(y, refs):
    for p, r in enumerate(refs):
        r[...] = y[:, p * PW:(p + 1) * PW]


def _h0_kernel(x_ref, w_ref, b_ref, oa_ref, ob_ref, oc_ref):
    y = jnp.maximum(_dotT(x_ref[...], w_ref[...]) + b_ref[...], 0.0)
    _split3(y, (oa_ref, ob_ref, oc_ref))


def _h0_pass(f_bonds, wi, wib, tile):
    pc = jax.ShapeDtypeStruct((NB, PW), _f32)
    return pl.pallas_call(
        _h0_kernel,
        grid=(NB // tile,),
        in_specs=[pl.BlockSpec((tile, BF), lambda i: (i, 0)),
                  pl.BlockSpec((HP, BF), lambda i: (0, 0)),
                  pl.BlockSpec((1, HP), lambda i: (0, 0))],
        out_specs=[pl.BlockSpec((tile, PW), lambda i: (i, 0))] * 3,
        out_shape=[pc, pc, pc],
    )(f_bonds, wi, wib)


def _t_kernel(sa_ref, sb_ref, sc0_ref, sc1_ref, wka_ref, wkb_ref, wkc_ref,
              b_ref, t_ref):
    t_ref[...] = (_dotT(sa_ref[...], wka_ref[...])
                  + _dotT(sb_ref[...], wkb_ref[...])
                  + _dotT(sc0_ref[...] + sc1_ref[...], wkc_ref[...])
                  + b_ref[...])


def _t_pass(sa, sb, sc0, sc1, wka, wkb, wkc, wmb, tile):
    return pl.pallas_call(
        _t_kernel,
        grid=(NA // tile,),
        in_specs=[pl.BlockSpec((tile, PW), lambda i: (i, 0))] * 4
                 + [pl.BlockSpec((HP, PW), lambda i: (0, 0))] * 3
                 + [pl.BlockSpec((1, HP), lambda i: (0, 0))],
        out_specs=pl.BlockSpec((tile, HP), lambda i: (i, 0)),
        out_shape=jax.ShapeDtypeStruct((NA, HP), _f32),
    )(sa, sb, sc0, sc1, wka, wkb, wkc, wmb)


def _msg_kernel(ha_ref, hb_ref, hc_ref, g_ref, wka_ref, wkb_ref, wkc_ref,
                oa_ref, ob_ref, oc_ref):
    r = (_dotT(ha_ref[...], wka_ref[...]) + _dotT(hb_ref[...], wkb_ref[...])
         + _dotT(hc_ref[...], wkc_ref[...]))
    y = jnp.maximum(g_ref[...] - r, 0.0)
    _split3(y, (oa_ref, ob_ref, oc_ref))


def _msg_pass(ha, hb, hc, g, wka, wkb, wkc, tile):
    pc = jax.ShapeDtypeStruct((NB, PW), _f32)
    return pl.pallas_call(
        _msg_kernel,
        grid=(NB // tile,),
        in_specs=[pl.BlockSpec((tile, PW), lambda i: (i, 0))] * 3
                 + [pl.BlockSpec((tile, HP), lambda i: (i, 0))]
                 + [pl.BlockSpec((HP, PW), lambda i: (0, 0))] * 3,
        out_specs=[pl.BlockSpec((tile, PW), lambda i: (i, 0))] * 3,
        out_shape=[pc, pc, pc],
    )(ha, hb, hc, g, wka, wkb, wkc)


def _msg_edge_kernel(ha_ref, hb_ref, hc_ref, g_ref, wka_ref, wkb_ref, wkc_ref,
                     ew_ref, eb_ref, oa_ref, ob_ref, oc_ref, e_ref):
    r = (_dotT(ha_ref[...], wka_ref[...]) + _dotT(hb_ref[...], wkb_ref[...])
         + _dotT(hc_ref[...], wkc_ref[...]))
    y = jnp.maximum(g_ref[...] - r, 0.0)
    _split3(y, (oa_ref, ob_ref, oc_ref))
    e_ref[...] = _dotT(y, ew_ref[...]) + eb_ref[...]


def _msg_edge_pass(ha, hb, hc, g, wka, wkb, wkc, ew, eb, tile):
    pc = jax.ShapeDtypeStruct((NB, PW), _f32)
    return pl.pallas_call(
        _msg_edge_kernel,
        grid=(NB // tile,),
        in_specs=[pl.BlockSpec((tile, PW), lambda i: (i, 0))] * 3
                 + [pl.BlockSpec((tile, HP), lambda i: (i, 0))]
                 + [pl.BlockSpec((HP, PW), lambda i: (0, 0))] * 3
                 + [pl.BlockSpec((EF, HP), lambda i: (0, 0)),
                    pl.BlockSpec((1, EF), lambda i: (0, 0))],
        out_specs=[pl.BlockSpec((tile, PW), lambda i: (i, 0))] * 3
                  + [pl.BlockSpec((tile, EF), lambda i: (i, 0))],
        out_shape=[pc, pc, pc, jax.ShapeDtypeStruct((NB, EF), _f32)],
    )(ha, hb, hc, g, wka, wkb, wkc, ew, eb)


_ATILE = 400
_MTILE = _ATILE // MOL


def _atoms_kernel(fa_ref, sa_ref, sb_ref, sc0_ref, sc1_ref, wa_ref, wab_ref,
                  nw_ref, nb_ref, np_ref, mol_ref):
    s3 = jnp.concatenate(
        [sa_ref[...], sb_ref[...], sc0_ref[...] + sc1_ref[...]], axis=1)
    ah = jnp.maximum(_dotT(fa_ref[...], wa_ref[...]) + wab_ref[...] + s3, 0.0)
    np_ref[...] = _dotT(ah, nw_ref[...]) + nb_ref[...]
    m_ids = lax.broadcasted_iota(jnp.int32, (_MTILE, _ATILE), 0)
    a_ids = lax.broadcasted_iota(jnp.int32, (_MTILE, _ATILE), 1) // MOL
    pool = (m_ids == a_ids).astype(_f32)
    mol_ref[...] = lax.dot_general(pool, ah, (((1,), (0,)), ((), ())),
                                   preferred_element_type=_f32,
                                   precision=_PREC)


def _atoms_pass(f_atoms, sa, sb, sc0, sc1, wa, wab, nw, nb):
    return pl.pallas_call(
        _atoms_kernel,
        grid=(NA // _ATILE,),
        in_specs=[pl.BlockSpec((_ATILE, AF), lambda i: (i, 0))]
                 + [pl.BlockSpec((_ATILE, PW), lambda i: (i, 0))] * 4
                 + [pl.BlockSpec((HP, AF), lambda i: (0, 0)),
                    pl.BlockSpec((1, HP), lambda i: (0, 0)),
                    pl.BlockSpec((AF, HP), lambda i: (0, 0)),
                    pl.BlockSpec((1, AF), lambda i: (0, 0))],
        out_specs=[pl.BlockSpec((_ATILE, AF), lambda i: (i, 0)),
                   pl.BlockSpec((_MTILE, HP), lambda i: (i, 0))],
        out_shape=[jax.ShapeDtypeStruct((NA, AF), _f32),
                   jax.ShapeDtypeStruct((NM, HP), _f32)],
    )(f_atoms, sa, sb, sc0, sc1, wa, wab, nw, nb)


def _graph_kernel(mol_ref, g1_ref, g1b_ref, g2_ref, g2b_ref, o_ref):
    gh = jnp.maximum(_dotT(mol_ref[...], g1_ref[...]) + g1b_ref[...], 0.0)
    o_ref[...] = lax.dot_general(g2_ref[...], gh, (((1,), (1,)), ((), ())),
                                 preferred_element_type=_f32,
                                 precision=_PREC) + g2b_ref[...]


def _graph_pass(mol, g1, g1b, g2, g2b):
    return pl.pallas_call(
        _graph_kernel,
        grid=(1,),
        in_specs=[pl.BlockSpec((NM, HP), lambda i: (0, 0)),
                  pl.BlockSpec((HP, HP), lambda i: (0, 0)),
                  pl.BlockSpec((1, HP), lambda i: (0, 0)),
                  pl.BlockSpec((1, HP), lambda i: (0, 0)),
                  pl.BlockSpec((1, 1), lambda i: (0, 0))],
        out_specs=pl.BlockSpec((1, NM), lambda i: (0, 0)),
        out_shape=jax.ShapeDtypeStruct((1, NM), _f32),
    )(mol, g1, g1b, g2, g2b)


def _pad2(a, rows, cols):
    return jnp.pad(a, ((0, rows - a.shape[0]), (0, cols - a.shape[1])))


def kernel(f_atoms, f_bonds, b2a, a_scope, Wi_w, Wi_b, Wm_w, Wm_b, Wa_w, Wa_b,
           node_w, node_b, edge_w, edge_b, g1_w, g1_b, g2_w, g2_b):
    b2a = b2a.astype(jnp.int32)
    b2a_rev = b2a.reshape(-1, 2)[:, ::-1].reshape(-1)

    wi = _pad2(Wi_w, HP, BF)
    wib = _pad2(Wi_b[None, :], 1, HP)
    wm = _pad2(Wm_w, HP, HP)
    wk = tuple(wm[:, p * PW:(p + 1) * PW] for p in range(NP))
    wmb = _pad2(Wm_b[None, :], 1, HP)
    wa = _pad2(Wa_w, HP, AF)
    wab = _pad2(Wa_b[None, :], 1, HP)
    nw = _pad2(node_w, AF, HP)
    nb = node_b[None, :]
    ew = _pad2(edge_w, EF, HP)
    eb = edge_b[None, :]
    g1 = _pad2(g1_w, HP, HP)
    g1b = _pad2(g1_b[None, :], 1, HP)
    g2 = _pad2(g2_w, 1, HP)
    g2b = g2_b[None, :]
    zrows = jnp.zeros((_RPS, PW), _f32)

    ha, hb, hc = _h0_pass(f_bonds, wi, wib, tile=512)

    idx_pairs = ((b2a, b2a_rev), (b2a_rev, b2a), (b2a, b2a_rev))
    edge_rev = None
    for d in range(3):
        sc_idx, g_idx = idx_pairs[d]
        sa, sb, sc0, sc1 = _sc_scatter3(ha, hb, hc, sc_idx, zrows)
        t = _t_pass(sa, sb, sc0, sc1, *wk, wmb, tile=2000)
        g = _sc_gather(t, g_idx)
        if d < 2:
            ha, hb, hc = _msg_pass(ha, hb, hc, g, *wk, tile=512)
        else:
            ha, hb, hc, edge_rev = _msg_edge_pass(ha, hb, hc, g, *wk,
                                                  ew, eb, tile=512)

    sa, sb, sc0, sc1 = _sc_scatter3(ha, hb, hc, b2a_rev, zrows)
    node_pred, mol = _atoms_pass(f_atoms, sa, sb, sc0, sc1, wa, wab, nw, nb)
    graph = _graph_pass(mol, g1, g1b, g2, g2b)

    edge_pred = edge_rev.reshape(-1, 2, EF)[:, ::-1].reshape(-1, EF)
    return node_pred, edge_pred, graph[0]


# speedup vs baseline: 1.6495x; 1.6298x over previous
import functools

import jax
import jax.numpy as jnp
from jax import lax
from jax.experimental import pallas as pl
from jax.experimental.pallas import tpu as pltpu
from jax.experimental.pallas import tpu_sc as plsc

NB = 320000
NA = 10000
PW = 128
NP = 3
HP = PW * NP
NM = 400
MOL = 25
AF = 128
BF = 144
EF = 16

_NC, _NS = 2, 16
_NW = _NC * _NS
_SCH = 80
NAP = 10240
_RPS = NAP // _NS

_PREC = jax.lax.Precision.DEFAULT
_f32 = jnp.float32


def _sc_scatter3(ha, hb, hc, idx2, zrows):
    mesh = plsc.VectorSubcoreMesh(core_axis_name="c", subcore_axis_name="s")
    out = jax.ShapeDtypeStruct((NAP, PW), _f32)

    @functools.partial(
        pl.kernel,
        out_type=[out, out, out, out],
        mesh=mesh,
        scratch_types=[
            pltpu.VMEM_SHARED((NAP, PW), _f32),
            pltpu.VMEM((NB // _NW // _SCH, _SCH), jnp.int32),
            pltpu.VMEM((_SCH, PW), _f32),
            pltpu.VMEM((_SCH, PW), _f32),
            pltpu.SemaphoreType.DMA,
            pltpu.SemaphoreType.DMA,
        ],
    )
    def k(a_hbm, b_hbm, c_hbm, idx3_hbm, z_hbm, sa, sb, sc0, sc1,
          acc, idxb, bufa, bufb, sema, semb):
        c = lax.axis_index("c")
        s = lax.axis_index("s")
        row0 = s * _RPS
        npw = NB // _NW // _SCH

        def start(x_hbm, bond0, j, buf, sem):
            pltpu.async_copy(x_hbm.at[pl.ds(bond0 + j * _SCH, _SCH)], buf, sem)

        def wait(x_hbm, bond0, buf, sem):
            pltpu.make_async_copy(x_hbm.at[pl.ds(bond0, _SCH)], buf, sem).wait()

        def scan(x_hbm, w):
            bond0 = w * (NB // _NW)
            pltpu.sync_copy(idx3_hbm.at[w], idxb)
            start(x_hbm, bond0, 0, bufa, sema)

            @pl.loop(0, npw - 1, step=2)
            def _(j):
                start(x_hbm, bond0, j + 1, bufb, semb)
                wait(x_hbm, bond0, bufa, sema)
                pltpu.sync_copy(bufa, acc.at[idxb.at[j]], add=True)
                start(x_hbm, bond0, j + 2, bufa, sema)
                wait(x_hbm, bond0, bufb, semb)
                pltpu.sync_copy(bufb, acc.at[idxb.at[j + 1]], add=True)

            wait(x_hbm, bond0, bufa, sema)
            pltpu.sync_copy(bufa, acc.at[idxb.at[npw - 1]], add=True)

        pltpu.sync_copy(z_hbm, acc.at[pl.ds(row0, _RPS)])
        plsc.subcore_barrier()

        @pl.when(c == 0)
        def _():
            scan(a_hbm, 2 * s)
            scan(a_hbm, 2 * s + 1)

        @pl.when(c == 1)
        def _():
            scan(b_hbm, 2 * s)
            scan(b_hbm, 2 * s + 1)

        plsc.subcore_barrier()

        @pl.when(c == 0)
        def _():
            pltpu.sync_copy(acc.at[pl.ds(row0, _RPS)], sa.at[pl.ds(row0, _RPS)])

        @pl.when(c == 1)
        def _():
            pltpu.sync_copy(acc.at[pl.ds(row0, _RPS)], sb.at[pl.ds(row0, _RPS)])

        pltpu.sync_copy(z_hbm, acc.at[pl.ds(row0, _RPS)])
        plsc.subcore_barrier()
        scan(c_hbm, c * _NS + s)
        plsc.subcore_barrier()

        @pl.when(c == 0)
        def _():
            pltpu.sync_copy(acc.at[pl.ds(row0, _RPS)], sc0.at[pl.ds(row0, _RPS)])

        @pl.when(c == 1)
        def _():
            pltpu.sync_copy(acc.at[pl.ds(row0, _RPS)], sc1.at[pl.ds(row0, _RPS)])

    return k(ha, hb, hc, idx2, zrows)


def _sc_gather(t, idx2):
    mesh = plsc.VectorSubcoreMesh(core_axis_name="c", subcore_axis_name="s")
    npw = NB // _NW // _SCH

    @functools.partial(
        pl.kernel,
        out_type=jax.ShapeDtypeStruct((NB, HP), _f32),
        mesh=mesh,
        scratch_types=[
            pltpu.VMEM((npw, _SCH), jnp.int32),
            pltpu.VMEM((_SCH, HP), _f32),
            pltpu.VMEM((_SCH, HP), _f32),
            pltpu.SemaphoreType.DMA,
            pltpu.SemaphoreType.DMA,
        ],
    )
    def k(t_hbm, idx3_hbm, g_hbm, idxb, bufa, bufb, sema, semb):
        wid = lax.axis_index("c") * _NS + lax.axis_index("s")
        base = wid * (NB // _NW)
        pltpu.sync_copy(idx3_hbm.at[wid], idxb)

        def gstart(j, buf, sem):
            pltpu.async_copy(t_hbm.at[idxb.at[j]], buf, sem)

        def gwait(buf, sem):
            pltpu.make_async_copy(t_hbm.at[idxb.at[0]], buf, sem).wait()

        def wout(j, buf):
            pltpu.sync_copy(buf, g_hbm.at[pl.ds(base + j * _SCH, _SCH)])

        gstart(0, bufa, sema)

        @pl.loop(0, npw - 1, step=2)
        def _(j):
            gstart(j + 1, bufb, semb)
            gwait(bufa, sema)
            wout(j, bufa)
            gstart(j + 2, bufa, sema)
            gwait(bufb, semb)
            wout(j + 1, bufb)

        gwait(bufa, sema)
        wout(npw - 1, bufa)

    return k(t, idx2)


def _dotT(x, w):
    return lax.dot_general(x, w, (((1,), (1,)), ((), ())),
                           preferred_element_type=_f32, precision=_PREC)


def _split3(y, refs):
    for p, r in enumerate(refs):
        r[...] = y[:, p * PW:(p + 1) * PW]


def _h0_kernel(x_ref, w_ref, b_ref, oa_ref, ob_ref, oc_ref):
    y = jnp.maximum(_dotT(x_ref[...], w_ref[...]) + b_ref[...], 0.0)
    _split3(y, (oa_ref, ob_ref, oc_ref))


def _h0_pass(f_bonds, wi, wib, tile):
    pc = jax.ShapeDtypeStruct((NB, PW), _f32)
    return pl.pallas_call(
        _h0_kernel,
        grid=(NB // tile,),
        in_specs=[pl.BlockSpec((tile, BF), lambda i: (i, 0)),
                  pl.BlockSpec((HP, BF), lambda i: (0, 0)),
                  pl.BlockSpec((1, HP), lambda i: (0, 0))],
        out_specs=[pl.BlockSpec((tile, PW), lambda i: (i, 0))] * 3,
        out_shape=[pc, pc, pc],
    )(f_bonds, wi, wib)


def _t_kernel(sa_ref, sb_ref, sc0_ref, sc1_ref, wka_ref, wkb_ref, wkc_ref,
              b_ref, t_ref):
    t_ref[...] = (_dotT(sa_ref[...], wka_ref[...])
                  + _dotT(sb_ref[...], wkb_ref[...])
                  + _dotT(sc0_ref[...] + sc1_ref[...], wkc_ref[...])
                  + b_ref[...])


def _t_pass(sa, sb, sc0, sc1, wka, wkb, wkc, wmb, tile):
    return pl.pallas_call(
        _t_kernel,
        grid=(NA // tile,),
        in_specs=[pl.BlockSpec((tile, PW), lambda i: (i, 0))] * 4
                 + [pl.BlockSpec((HP, PW), lambda i: (0, 0))] * 3
                 + [pl.BlockSpec((1, HP), lambda i: (0, 0))],
        out_specs=pl.BlockSpec((tile, HP), lambda i: (i, 0)),
        out_shape=jax.ShapeDtypeStruct((NA, HP), _f32),
    )(sa, sb, sc0, sc1, wka, wkb, wkc, wmb)


def _msg_kernel(ha_ref, hb_ref, hc_ref, g_ref, wka_ref, wkb_ref, wkc_ref,
                oa_ref, ob_ref, oc_ref):
    r = (_dotT(ha_ref[...], wka_ref[...]) + _dotT(hb_ref[...], wkb_ref[...])
         + _dotT(hc_ref[...], wkc_ref[...]))
    y = jnp.maximum(g_ref[...] - r, 0.0)
    _split3(y, (oa_ref, ob_ref, oc_ref))


def _msg_pass(ha, hb, hc, g, wka, wkb, wkc, tile):
    pc = jax.ShapeDtypeStruct((NB, PW), _f32)
    return pl.pallas_call(
        _msg_kernel,
        grid=(NB // tile,),
        in_specs=[pl.BlockSpec((tile, PW), lambda i: (i, 0))] * 3
                 + [pl.BlockSpec((tile, HP), lambda i: (i, 0))]
                 + [pl.BlockSpec((HP, PW), lambda i: (0, 0))] * 3,
        out_specs=[pl.BlockSpec((tile, PW), lambda i: (i, 0))] * 3,
        out_shape=[pc, pc, pc],
    )(ha, hb, hc, g, wka, wkb, wkc)


def _msg_edge_kernel(ha_ref, hb_ref, hc_ref, g_ref, wka_ref, wkb_ref, wkc_ref,
                     ew_ref, eb_ref, oa_ref, ob_ref, oc_ref, e_ref):
    r = (_dotT(ha_ref[...], wka_ref[...]) + _dotT(hb_ref[...], wkb_ref[...])
         + _dotT(hc_ref[...], wkc_ref[...]))
    y = jnp.maximum(g_ref[...] - r, 0.0)
    _split3(y, (oa_ref, ob_ref, oc_ref))
    e_ref[...] = _dotT(y, ew_ref[...]) + eb_ref[...]


def _msg_edge_pass(ha, hb, hc, g, wka, wkb, wkc, ew, eb, tile):
    pc = jax.ShapeDtypeStruct((NB, PW), _f32)
    return pl.pallas_call(
        _msg_edge_kernel,
        grid=(NB // tile,),
        in_specs=[pl.BlockSpec((tile, PW), lambda i: (i, 0))] * 3
                 + [pl.BlockSpec((tile, HP), lambda i: (i, 0))]
                 + [pl.BlockSpec((HP, PW), lambda i: (0, 0))] * 3
                 + [pl.BlockSpec((EF, HP), lambda i: (0, 0)),
                    pl.BlockSpec((1, EF), lambda i: (0, 0))],
        out_specs=[pl.BlockSpec((tile, PW), lambda i: (i, 0))] * 3
                  + [pl.BlockSpec((tile, EF), lambda i: (i, 0))],
        out_shape=[pc, pc, pc, jax.ShapeDtypeStruct((NB, EF), _f32)],
    )(ha, hb, hc, g, wka, wkb, wkc, ew, eb)


_ATILE = 400
_MTILE = _ATILE // MOL


def _atoms_kernel(fa_ref, sa_ref, sb_ref, sc0_ref, sc1_ref, wa_ref, wab_ref,
                  nw_ref, nb_ref, np_ref, mol_ref):
    s3 = jnp.concatenate(
        [sa_ref[...], sb_ref[...], sc0_ref[...] + sc1_ref[...]], axis=1)
    ah = jnp.maximum(_dotT(fa_ref[...], wa_ref[...]) + wab_ref[...] + s3, 0.0)
    np_ref[...] = _dotT(ah, nw_ref[...]) + nb_ref[...]
    m_ids = lax.broadcasted_iota(jnp.int32, (_MTILE, _ATILE), 0)
    a_ids = lax.broadcasted_iota(jnp.int32, (_MTILE, _ATILE), 1) // MOL
    pool = (m_ids == a_ids).astype(_f32)
    mol_ref[...] = lax.dot_general(pool, ah, (((1,), (0,)), ((), ())),
                                   preferred_element_type=_f32,
                                   precision=_PREC)


def _atoms_pass(f_atoms, sa, sb, sc0, sc1, wa, wab, nw, nb):
    return pl.pallas_call(
        _atoms_kernel,
        grid=(NA // _ATILE,),
        in_specs=[pl.BlockSpec((_ATILE, AF), lambda i: (i, 0))]
                 + [pl.BlockSpec((_ATILE, PW), lambda i: (i, 0))] * 4
                 + [pl.BlockSpec((HP, AF), lambda i: (0, 0)),
                    pl.BlockSpec((1, HP), lambda i: (0, 0)),
                    pl.BlockSpec((AF, HP), lambda i: (0, 0)),
                    pl.BlockSpec((1, AF), lambda i: (0, 0))],
        out_specs=[pl.BlockSpec((_ATILE, AF), lambda i: (i, 0)),
                   pl.BlockSpec((_MTILE, HP), lambda i: (i, 0))],
        out_shape=[jax.ShapeDtypeStruct((NA, AF), _f32),
                   jax.ShapeDtypeStruct((NM, HP), _f32)],
    )(f_atoms, sa, sb, sc0, sc1, wa, wab, nw, nb)


def _graph_kernel(mol_ref, g1_ref, g1b_ref, g2_ref, g2b_ref, o_ref):
    gh = jnp.maximum(_dotT(mol_ref[...], g1_ref[...]) + g1b_ref[...], 0.0)
    o_ref[...] = lax.dot_general(g2_ref[...], gh, (((1,), (1,)), ((), ())),
                                 preferred_element_type=_f32,
                                 precision=_PREC) + g2b_ref[...]


def _graph_pass(mol, g1, g1b, g2, g2b):
    return pl.pallas_call(
        _graph_kernel,
        grid=(1,),
        in_specs=[pl.BlockSpec((NM, HP), lambda i: (0, 0)),
                  pl.BlockSpec((HP, HP), lambda i: (0, 0)),
                  pl.BlockSpec((1, HP), lambda i: (0, 0)),
                  pl.BlockSpec((1, HP), lambda i: (0, 0)),
                  pl.BlockSpec((1, 1), lambda i: (0, 0))],
        out_specs=pl.BlockSpec((1, NM), lambda i: (0, 0)),
        out_shape=jax.ShapeDtypeStruct((1, NM), _f32),
    )(mol, g1, g1b, g2, g2b)


def _pad2(a, rows, cols):
    return jnp.pad(a, ((0, rows - a.shape[0]), (0, cols - a.shape[1])))


def kernel(f_atoms, f_bonds, b2a, a_scope, Wi_w, Wi_b, Wm_w, Wm_b, Wa_w, Wa_b,
           node_w, node_b, edge_w, edge_b, g1_w, g1_b, g2_w, g2_b):
    b2a = b2a.astype(jnp.int32)
    b2a_rev = b2a.reshape(-1, 2)[:, ::-1].reshape(-1)
    b2a2 = b2a.reshape(_NW, NB // _NW // _SCH, _SCH)
    b2a_rev2 = b2a_rev.reshape(_NW, NB // _NW // _SCH, _SCH)

    wi = _pad2(Wi_w, HP, BF)
    wib = _pad2(Wi_b[None, :], 1, HP)
    wm = _pad2(Wm_w, HP, HP)
    wk = tuple(wm[:, p * PW:(p + 1) * PW] for p in range(NP))
    wmb = _pad2(Wm_b[None, :], 1, HP)
    wa = _pad2(Wa_w, HP, AF)
    wab = _pad2(Wa_b[None, :], 1, HP)
    nw = _pad2(node_w, AF, HP)
    nb = node_b[None, :]
    ew = _pad2(edge_w, EF, HP)
    eb = edge_b[None, :]
    g1 = _pad2(g1_w, HP, HP)
    g1b = _pad2(g1_b[None, :], 1, HP)
    g2 = _pad2(g2_w, 1, HP)
    g2b = g2_b[None, :]
    zrows = jnp.zeros((_RPS, PW), _f32)

    ha, hb, hc = _h0_pass(f_bonds, wi, wib, tile=512)

    idx_pairs = ((b2a2, b2a_rev2), (b2a_rev2, b2a2), (b2a2, b2a_rev2))
    edge_rev = None
    for d in range(3):
        sc_idx, g_idx = idx_pairs[d]
        sa, sb, sc0, sc1 = _sc_scatter3(ha, hb, hc, sc_idx, zrows)
        t = _t_pass(sa, sb, sc0, sc1, *wk, wmb, tile=2000)
        g = _sc_gather(t, g_idx)
        if d < 2:
            ha, hb, hc = _msg_pass(ha, hb, hc, g, *wk, tile=512)
        else:
            ha, hb, hc, edge_rev = _msg_edge_pass(ha, hb, hc, g, *wk,
                                                  ew, eb, tile=512)

    sa, sb, sc0, sc1 = _sc_scatter3(ha, hb, hc, b2a_rev2, zrows)
    node_pred, mol = _atoms_pass(f_atoms, sa, sb, sc0, sc1, wa, wab, nw, nb)
    graph = _graph_pass(mol, g1, g1b, g2, g2b)

    edge_pred = edge_rev.reshape(-1, 2, EF)[:, ::-1].reshape(-1, EF)
    return node_pred, edge_pred, graph[0]


# speedup vs baseline: 1.8987x; 1.1511x over previous
import functools

import jax
import jax.numpy as jnp
from jax import lax
from jax.experimental import pallas as pl
from jax.experimental.pallas import tpu as pltpu
from jax.experimental.pallas import tpu_sc as plsc

NB = 320000
NA = 10000
PW = 128
NP = 3
HP = PW * NP
NM = 400
MOL = 25
AF = 128
BF = 144
EF = 16

_NC, _NS = 2, 16
_NW = _NC * _NS
_SCH = 80
NAP = 10240
_RPS = NAP // _NS

_PREC = jax.lax.Precision.DEFAULT
_f32 = jnp.float32


def _sc_scatter3(ha, hb, hc, idx2, zrows):
    mesh = plsc.VectorSubcoreMesh(core_axis_name="c", subcore_axis_name="s")
    out = jax.ShapeDtypeStruct((NAP, PW), _f32)

    @functools.partial(
        pl.kernel,
        out_type=[out, out, out, out],
        mesh=mesh,
        scratch_types=[
            pltpu.VMEM_SHARED((NAP, PW), _f32),
            pltpu.VMEM((_SCH,), jnp.int32),
            pltpu.VMEM((_SCH,), jnp.int32),
            pltpu.VMEM((_SCH, PW), _f32),
            pltpu.VMEM((_SCH, PW), _f32),
            pltpu.SemaphoreType.DMA,
            pltpu.SemaphoreType.DMA,
        ],
    )
    def k(a_hbm, b_hbm, c_hbm, idx_hbm, z_hbm, sa, sb, sc0, sc1,
          acc, idxa, idxb, bufa, bufb, sema, semb):
        c = lax.axis_index("c")
        s = lax.axis_index("s")
        row0 = s * _RPS
        npw = NB // _NW // _SCH

        def start(x_hbm, bond0, j, buf, ib, sem):
            pltpu.async_copy(x_hbm.at[pl.ds(bond0 + j * _SCH, _SCH)], buf, sem)
            pltpu.async_copy(idx_hbm.at[pl.ds(bond0 + j * _SCH, _SCH)], ib, sem)

        def wait(x_hbm, bond0, buf, ib, sem):
            pltpu.make_async_copy(x_hbm.at[pl.ds(bond0, _SCH)], buf, sem).wait()
            pltpu.make_async_copy(idx_hbm.at[pl.ds(bond0, _SCH)], ib, sem).wait()

        def scan(x_hbm, w):
            bond0 = w * (NB // _NW)
            start(x_hbm, bond0, 0, bufa, idxa, sema)

            @pl.loop(0, npw - 1, step=2)
            def _(j):
                start(x_hbm, bond0, j + 1, bufb, idxb, semb)
                wait(x_hbm, bond0, bufa, idxa, sema)
                pltpu.sync_copy(bufa, acc.at[idxa], add=True)
                start(x_hbm, bond0, j + 2, bufa, idxa, sema)
                wait(x_hbm, bond0, bufb, idxb, semb)
                pltpu.sync_copy(bufb, acc.at[idxb], add=True)

            wait(x_hbm, bond0, bufa, idxa, sema)
            pltpu.sync_copy(bufa, acc.at[idxa], add=True)

        pltpu.sync_copy(z_hbm, acc.at[pl.ds(row0, _RPS)])
        plsc.subcore_barrier()

        @pl.when(c == 0)
        def _():
            scan(a_hbm, 2 * s)
            scan(a_hbm, 2 * s + 1)

        @pl.when(c == 1)
        def _():
            scan(b_hbm, 2 * s)
            scan(b_hbm, 2 * s + 1)

        plsc.subcore_barrier()

        @pl.when(c == 0)
        def _():
            pltpu.sync_copy(acc.at[pl.ds(row0, _RPS)], sa.at[pl.ds(row0, _RPS)])

        @pl.when(c == 1)
        def _():
            pltpu.sync_copy(acc.at[pl.ds(row0, _RPS)], sb.at[pl.ds(row0, _RPS)])

        pltpu.sync_copy(z_hbm, acc.at[pl.ds(row0, _RPS)])
        plsc.subcore_barrier()
        scan(c_hbm, c * _NS + s)
        plsc.subcore_barrier()

        @pl.when(c == 0)
        def _():
            pltpu.sync_copy(acc.at[pl.ds(row0, _RPS)], sc0.at[pl.ds(row0, _RPS)])

        @pl.when(c == 1)
        def _():
            pltpu.sync_copy(acc.at[pl.ds(row0, _RPS)], sc1.at[pl.ds(row0, _RPS)])

    return k(ha, hb, hc, idx2, zrows)


def _sc_gather(t, idx):
    mesh = plsc.VectorSubcoreMesh(core_axis_name="c", subcore_axis_name="s")
    npw = NB // _NW // _SCH

    @functools.partial(
        pl.kernel,
        out_type=jax.ShapeDtypeStruct((NB, HP), _f32),
        mesh=mesh,
        scratch_types=[
            pltpu.VMEM((NB // _NW,), jnp.int32),
            pltpu.VMEM((_SCH, HP), _f32),
            pltpu.VMEM((_SCH, HP), _f32),
            pltpu.SemaphoreType.DMA,
            pltpu.SemaphoreType.DMA,
        ],
    )
    def k(t_hbm, idx_hbm, g_hbm, idxb, bufa, bufb, sema, semb):
        wid = lax.axis_index("c") * _NS + lax.axis_index("s")
        base = wid * (NB // _NW)
        pltpu.sync_copy(idx_hbm.at[pl.ds(base, NB // _NW)], idxb)

        def gstart(j, buf, sem):
            pltpu.async_copy(t_hbm.at[idxb.at[pl.ds(j * _SCH, _SCH)]], buf, sem)

        def gwait(buf, sem):
            pltpu.make_async_copy(t_hbm.at[idxb.at[pl.ds(0, _SCH)]], buf,
                                  sem).wait()

        def wout(j, buf):
            pltpu.sync_copy(buf, g_hbm.at[pl.ds(base + j * _SCH, _SCH)])

        gstart(0, bufa, sema)

        @pl.loop(0, npw - 1, step=2)
        def _(j):
            gstart(j + 1, bufb, semb)
            gwait(bufa, sema)
            wout(j, bufa)
            gstart(j + 2, bufa, sema)
            gwait(bufb, semb)
            wout(j + 1, bufb)

        gwait(bufa, sema)
        wout(npw - 1, bufa)

    return k(t, idx)


def _dotT(x, w):
    return lax.dot_general(x, w, (((1,), (1,)), ((), ())),
                           preferred_element_type=_f32, precision=_PREC)


def _split3(y, refs):
    for p, r in enumerate(refs):
        r[...] = y[:, p * PW:(p + 1) * PW]


def _h0_kernel(x_ref, w_ref, b_ref, oa_ref, ob_ref, oc_ref):
    y = jnp.maximum(_dotT(x_ref[...], w_ref[...]) + b_ref[...], 0.0)
    _split3(y, (oa_ref, ob_ref, oc_ref))


def _h0_pass(f_bonds, wi, wib, tile):
    pc = jax.ShapeDtypeStruct((NB, PW), _f32)
    return pl.pallas_call(
        _h0_kernel,
        grid=(NB // tile,),
        in_specs=[pl.BlockSpec((tile, BF), lambda i: (i, 0)),
                  pl.BlockSpec((HP, BF), lambda i: (0, 0)),
                  pl.BlockSpec((1, HP), lambda i: (0, 0))],
        out_specs=[pl.BlockSpec((tile, PW), lambda i: (i, 0))] * 3,
        out_shape=[pc, pc, pc],
    )(f_bonds, wi, wib)


def _t_kernel(sa_ref, sb_ref, sc0_ref, sc1_ref, wka_ref, wkb_ref, wkc_ref,
              b_ref, t_ref):
    t_ref[...] = (_dotT(sa_ref[...], wka_ref[...])
                  + _dotT(sb_ref[...], wkb_ref[...])
                  + _dotT(sc0_ref[...] + sc1_ref[...], wkc_ref[...])
                  + b_ref[...])


def _t_pass(sa, sb, sc0, sc1, wka, wkb, wkc, wmb, tile):
    return pl.pallas_call(
        _t_kernel,
        grid=(NA // tile,),
        in_specs=[pl.BlockSpec((tile, PW), lambda i: (i, 0))] * 4
                 + [pl.BlockSpec((HP, PW), lambda i: (0, 0))] * 3
                 + [pl.BlockSpec((1, HP), lambda i: (0, 0))],
        out_specs=pl.BlockSpec((tile, HP), lambda i: (i, 0)),
        out_shape=jax.ShapeDtypeStruct((NA, HP), _f32),
    )(sa, sb, sc0, sc1, wka, wkb, wkc, wmb)


def _msg_kernel(ha_ref, hb_ref, hc_ref, g_ref, wka_ref, wkb_ref, wkc_ref,
                oa_ref, ob_ref, oc_ref):
    r = (_dotT(ha_ref[...], wka_ref[...]) + _dotT(hb_ref[...], wkb_ref[...])
         + _dotT(hc_ref[...], wkc_ref[...]))
    y = jnp.maximum(g_ref[...] - r, 0.0)
    _split3(y, (oa_ref, ob_ref, oc_ref))


def _msg_pass(ha, hb, hc, g, wka, wkb, wkc, tile):
    pc = jax.ShapeDtypeStruct((NB, PW), _f32)
    return pl.pallas_call(
        _msg_kernel,
        grid=(NB // tile,),
        in_specs=[pl.BlockSpec((tile, PW), lambda i: (i, 0))] * 3
                 + [pl.BlockSpec((tile, HP), lambda i: (i, 0))]
                 + [pl.BlockSpec((HP, PW), lambda i: (0, 0))] * 3,
        out_specs=[pl.BlockSpec((tile, PW), lambda i: (i, 0))] * 3,
        out_shape=[pc, pc, pc],
    )(ha, hb, hc, g, wka, wkb, wkc)


def _msg_edge_kernel(ha_ref, hb_ref, hc_ref, g_ref, wka_ref, wkb_ref, wkc_ref,
                     ew_ref, eb_ref, oa_ref, ob_ref, oc_ref, e_ref):
    r = (_dotT(ha_ref[...], wka_ref[...]) + _dotT(hb_ref[...], wkb_ref[...])
         + _dotT(hc_ref[...], wkc_ref[...]))
    y = jnp.maximum(g_ref[...] - r, 0.0)
    _split3(y, (oa_ref, ob_ref, oc_ref))
    e_ref[...] = _dotT(y, ew_ref[...]) + eb_ref[...]


def _msg_edge_pass(ha, hb, hc, g, wka, wkb, wkc, ew, eb, tile):
    pc = jax.ShapeDtypeStruct((NB, PW), _f32)
    return pl.pallas_call(
        _msg_edge_kernel,
        grid=(NB // tile,),
        in_specs=[pl.BlockSpec((tile, PW), lambda i: (i, 0))] * 3
                 + [pl.BlockSpec((tile, HP), lambda i: (i, 0))]
                 + [pl.BlockSpec((HP, PW), lambda i: (0, 0))] * 3
                 + [pl.BlockSpec((EF, HP), lambda i: (0, 0)),
                    pl.BlockSpec((1, EF), lambda i: (0, 0))],
        out_specs=[pl.BlockSpec((tile, PW), lambda i: (i, 0))] * 3
                  + [pl.BlockSpec((tile, EF), lambda i: (i, 0))],
        out_shape=[pc, pc, pc, jax.ShapeDtypeStruct((NB, EF), _f32)],
    )(ha, hb, hc, g, wka, wkb, wkc, ew, eb)


_ATILE = 400
_MTILE = _ATILE // MOL


def _atoms_kernel(fa_ref, sa_ref, sb_ref, sc0_ref, sc1_ref, wa_ref, wab_ref,
                  nw_ref, nb_ref, np_ref, mol_ref):
    s3 = jnp.concatenate(
        [sa_ref[...], sb_ref[...], sc0_ref[...] + sc1_ref[...]], axis=1)
    ah = jnp.maximum(_dotT(fa_ref[...], wa_ref[...]) + wab_ref[...] + s3, 0.0)
    np_ref[...] = _dotT(ah, nw_ref[...]) + nb_ref[...]
    m_ids = lax.broadcasted_iota(jnp.int32, (_MTILE, _ATILE), 0)
    a_ids = lax.broadcasted_iota(jnp.int32, (_MTILE, _ATILE), 1) // MOL
    pool = (m_ids == a_ids).astype(_f32)
    mol_ref[...] = lax.dot_general(pool, ah, (((1,), (0,)), ((), ())),
                                   preferred_element_type=_f32,
                                   precision=_PREC)


def _atoms_pass(f_atoms, sa, sb, sc0, sc1, wa, wab, nw, nb):
    return pl.pallas_call(
        _atoms_kernel,
        grid=(NA // _ATILE,),
        in_specs=[pl.BlockSpec((_ATILE, AF), lambda i: (i, 0))]
                 + [pl.BlockSpec((_ATILE, PW), lambda i: (i, 0))] * 4
                 + [pl.BlockSpec((HP, AF), lambda i: (0, 0)),
                    pl.BlockSpec((1, HP), lambda i: (0, 0)),
                    pl.BlockSpec((AF, HP), lambda i: (0, 0)),
                    pl.BlockSpec((1, AF), lambda i: (0, 0))],
        out_specs=[pl.BlockSpec((_ATILE, AF), lambda i: (i, 0)),
                   pl.BlockSpec((_MTILE, HP), lambda i: (i, 0))],
        out_shape=[jax.ShapeDtypeStruct((NA, AF), _f32),
                   jax.ShapeDtypeStruct((NM, HP), _f32)],
    )(f_atoms, sa, sb, sc0, sc1, wa, wab, nw, nb)


def _graph_kernel(mol_ref, g1_ref, g1b_ref, g2_ref, g2b_ref, o_ref):
    gh = jnp.maximum(_dotT(mol_ref[...], g1_ref[...]) + g1b_ref[...], 0.0)
    o_ref[...] = lax.dot_general(g2_ref[...], gh, (((1,), (1,)), ((), ())),
                                 preferred_element_type=_f32,
                                 precision=_PREC) + g2b_ref[...]


def _graph_pass(mol, g1, g1b, g2, g2b):
    return pl.pallas_call(
        _graph_kernel,
        grid=(1,),
        in_specs=[pl.BlockSpec((NM, HP), lambda i: (0, 0)),
                  pl.BlockSpec((HP, HP), lambda i: (0, 0)),
                  pl.BlockSpec((1, HP), lambda i: (0, 0)),
                  pl.BlockSpec((1, HP), lambda i: (0, 0)),
                  pl.BlockSpec((1, 1), lambda i: (0, 0))],
        out_specs=pl.BlockSpec((1, NM), lambda i: (0, 0)),
        out_shape=jax.ShapeDtypeStruct((1, NM), _f32),
    )(mol, g1, g1b, g2, g2b)


def _pad2(a, rows, cols):
    return jnp.pad(a, ((0, rows - a.shape[0]), (0, cols - a.shape[1])))


def kernel(f_atoms, f_bonds, b2a, a_scope, Wi_w, Wi_b, Wm_w, Wm_b, Wa_w, Wa_b,
           node_w, node_b, edge_w, edge_b, g1_w, g1_b, g2_w, g2_b):
    b2a = b2a.astype(jnp.int32)
    even = (lax.iota(jnp.int32, NB) % 2) == 0
    b2a_rev = jnp.where(even, jnp.roll(b2a, -1), jnp.roll(b2a, 1))

    wi = _pad2(Wi_w, HP, BF)
    wib = _pad2(Wi_b[None, :], 1, HP)
    wm = _pad2(Wm_w, HP, HP)
    wk = tuple(wm[:, p * PW:(p + 1) * PW] for p in range(NP))
    wmb = _pad2(Wm_b[None, :], 1, HP)
    wa = _pad2(Wa_w, HP, AF)
    wab = _pad2(Wa_b[None, :], 1, HP)
    nw = _pad2(node_w, AF, HP)
    nb = node_b[None, :]
    ew = _pad2(edge_w, EF, HP)
    eb = edge_b[None, :]
    g1 = _pad2(g1_w, HP, HP)
    g1b = _pad2(g1_b[None, :], 1, HP)
    g2 = _pad2(g2_w, 1, HP)
    g2b = g2_b[None, :]
    zrows = jnp.zeros((_RPS, PW), _f32)

    ha, hb, hc = _h0_pass(f_bonds, wi, wib, tile=512)

    idx_pairs = ((b2a, b2a_rev), (b2a_rev, b2a), (b2a, b2a_rev))
    edge_rev = None
    for d in range(3):
        sc_idx, g_idx = idx_pairs[d]
        sa, sb, sc0, sc1 = _sc_scatter3(ha, hb, hc, sc_idx, zrows)
        t = _t_pass(sa, sb, sc0, sc1, *wk, wmb, tile=2000)
        g = _sc_gather(t, g_idx)
        if d < 2:
            ha, hb, hc = _msg_pass(ha, hb, hc, g, *wk, tile=512)
        else:
            ha, hb, hc, edge_rev = _msg_edge_pass(ha, hb, hc, g, *wk,
                                                  ew, eb, tile=512)

    sa, sb, sc0, sc1 = _sc_scatter3(ha, hb, hc, b2a_rev, zrows)
    node_pred, mol = _atoms_pass(f_atoms, sa, sb, sc0, sc1, wa, wab, nw, nb)
    graph = _graph_pass(mol, g1, g1b, g2, g2b)

    edge_pred = jnp.where(even[:, None], jnp.roll(edge_rev, -1, axis=0),
                          jnp.roll(edge_rev, 1, axis=0))
    return node_pred, edge_pred, graph[0]


# speedup vs baseline: 1.9389x; 1.0212x over previous
import functools

import jax
import jax.numpy as jnp
from jax import lax
from jax.experimental import pallas as pl
from jax.experimental.pallas import tpu as pltpu
from jax.experimental.pallas import tpu_sc as plsc

NB = 320000
NA = 10000
PW = 128
NP = 3
HP = PW * NP
NM = 400
MOL = 25
AF = 128
BF = 144
EF = 16

_NC, _NS = 2, 16
_NW = _NC * _NS
_SCH = 80
NAP = 10240
_RPS = NAP // _NS

_PREC = jax.lax.Precision.DEFAULT
_HI = jax.lax.Precision.HIGHEST
_f32 = jnp.float32


def _sc_scatter3(ha, hb, hc, idx2, zrows):
    mesh = plsc.VectorSubcoreMesh(core_axis_name="c", subcore_axis_name="s")
    out = jax.ShapeDtypeStruct((NAP, PW), _f32)

    @functools.partial(
        pl.kernel,
        out_type=[out, out, out, out],
        mesh=mesh,
        scratch_types=[
            pltpu.VMEM_SHARED((NAP, PW), _f32),
            pltpu.VMEM((_SCH,), jnp.int32),
            pltpu.VMEM((_SCH,), jnp.int32),
            pltpu.VMEM((_SCH, PW), _f32),
            pltpu.VMEM((_SCH, PW), _f32),
            pltpu.SemaphoreType.DMA,
            pltpu.SemaphoreType.DMA,
        ],
    )
    def k(a_hbm, b_hbm, c_hbm, idx_hbm, z_hbm, sa, sb, sc0, sc1,
          acc, idxa, idxb, bufa, bufb, sema, semb):
        c = lax.axis_index("c")
        s = lax.axis_index("s")
        row0 = s * _RPS
        npw = NB // _NW // _SCH

        def start(x_hbm, bond0, j, buf, ib, sem):
            pltpu.async_copy(x_hbm.at[pl.ds(bond0 + j * _SCH, _SCH)], buf, sem)
            pltpu.async_copy(idx_hbm.at[pl.ds(bond0 + j * _SCH, _SCH)], ib, sem)

        def wait(x_hbm, bond0, buf, ib, sem):
            pltpu.make_async_copy(x_hbm.at[pl.ds(bond0, _SCH)], buf, sem).wait()
            pltpu.make_async_copy(idx_hbm.at[pl.ds(bond0, _SCH)], ib, sem).wait()

        def scan(x_hbm, w):
            bond0 = w * (NB // _NW)
            start(x_hbm, bond0, 0, bufa, idxa, sema)

            @pl.loop(0, npw - 1, step=2)
            def _(j):
                start(x_hbm, bond0, j + 1, bufb, idxb, semb)
                wait(x_hbm, bond0, bufa, idxa, sema)
                pltpu.sync_copy(bufa, acc.at[idxa], add=True)
                start(x_hbm, bond0, j + 2, bufa, idxa, sema)
                wait(x_hbm, bond0, bufb, idxb, semb)
                pltpu.sync_copy(bufb, acc.at[idxb], add=True)

            wait(x_hbm, bond0, bufa, idxa, sema)
            pltpu.sync_copy(bufa, acc.at[idxa], add=True)

        pltpu.sync_copy(z_hbm, acc.at[pl.ds(row0, _RPS)])
        plsc.subcore_barrier()

        @pl.when(c == 0)
        def _():
            scan(a_hbm, 2 * s)
            scan(a_hbm, 2 * s + 1)

        @pl.when(c == 1)
        def _():
            scan(b_hbm, 2 * s)
            scan(b_hbm, 2 * s + 1)

        plsc.subcore_barrier()

        @pl.when(c == 0)
        def _():
            pltpu.sync_copy(acc.at[pl.ds(row0, _RPS)], sa.at[pl.ds(row0, _RPS)])

        @pl.when(c == 1)
        def _():
            pltpu.sync_copy(acc.at[pl.ds(row0, _RPS)], sb.at[pl.ds(row0, _RPS)])

        pltpu.sync_copy(z_hbm, acc.at[pl.ds(row0, _RPS)])
        plsc.subcore_barrier()
        scan(c_hbm, c * _NS + s)
        plsc.subcore_barrier()

        @pl.when(c == 0)
        def _():
            pltpu.sync_copy(acc.at[pl.ds(row0, _RPS)], sc0.at[pl.ds(row0, _RPS)])

        @pl.when(c == 1)
        def _():
            pltpu.sync_copy(acc.at[pl.ds(row0, _RPS)], sc1.at[pl.ds(row0, _RPS)])

    return k(ha, hb, hc, idx2, zrows)


def _sc_gather(t, idx):
    mesh = plsc.VectorSubcoreMesh(core_axis_name="c", subcore_axis_name="s")
    npw = NB // _NW // _SCH

    @functools.partial(
        pl.kernel,
        out_type=jax.ShapeDtypeStruct((NB, HP), _f32),
        mesh=mesh,
        scratch_types=[
            pltpu.VMEM((NB // _NW,), jnp.int32),
            pltpu.VMEM((_SCH, HP), _f32),
            pltpu.VMEM((_SCH, HP), _f32),
            pltpu.SemaphoreType.DMA,
            pltpu.SemaphoreType.DMA,
        ],
    )
    def k(t_hbm, idx_hbm, g_hbm, idxb, bufa, bufb, sema, semb):
        wid = lax.axis_index("c") * _NS + lax.axis_index("s")
        base = wid * (NB // _NW)
        pltpu.sync_copy(idx_hbm.at[pl.ds(base, NB // _NW)], idxb)

        def gstart(j, buf, sem):
            pltpu.async_copy(t_hbm.at[idxb.at[pl.ds(j * _SCH, _SCH)]], buf, sem)

        def gwait(buf, sem):
            pltpu.make_async_copy(t_hbm.at[idxb.at[pl.ds(0, _SCH)]], buf,
                                  sem).wait()

        def wout(j, buf):
            pltpu.sync_copy(buf, g_hbm.at[pl.ds(base + j * _SCH, _SCH)])

        gstart(0, bufa, sema)

        @pl.loop(0, npw - 1, step=2)
        def _(j):
            gstart(j + 1, bufb, semb)
            gwait(bufa, sema)
            wout(j, bufa)
            gstart(j + 2, bufa, sema)
            gwait(bufb, semb)
            wout(j + 1, bufb)

        gwait(bufa, sema)
        wout(npw - 1, bufa)

    return k(t, idx)


def _dotT(x, w, prec=_PREC):
    return lax.dot_general(x, w, (((1,), (1,)), ((), ())),
                           preferred_element_type=_f32, precision=prec)


def _split3(y, refs):
    for p, r in enumerate(refs):
        r[...] = y[:, p * PW:(p + 1) * PW]


def _h0_kernel(x_ref, w_ref, b_ref, oa_ref, ob_ref, oc_ref):
    y = jnp.maximum(_dotT(x_ref[...], w_ref[...]) + b_ref[...], 0.0)
    _split3(y, (oa_ref, ob_ref, oc_ref))


def _h0_pass(f_bonds, wi, wib, tile):
    pc = jax.ShapeDtypeStruct((NB, PW), _f32)
    return pl.pallas_call(
        _h0_kernel,
        grid=(NB // tile,),
        in_specs=[pl.BlockSpec((tile, BF), lambda i: (i, 0)),
                  pl.BlockSpec((HP, BF), lambda i: (0, 0)),
                  pl.BlockSpec((1, HP), lambda i: (0, 0))],
        out_specs=[pl.BlockSpec((tile, PW), lambda i: (i, 0))] * 3,
        out_shape=[pc, pc, pc],
    )(f_bonds, wi, wib)


def _s_kernel(sa_ref, sb_ref, sc0_ref, sc1_ref, s_ref):
    s_ref[...] = jnp.concatenate(
        [sa_ref[...], sb_ref[...], sc0_ref[...] + sc1_ref[...]], axis=1)


def _s_pass(sa, sb, sc0, sc1, tile):
    return pl.pallas_call(
        _s_kernel,
        grid=(NA // tile,),
        in_specs=[pl.BlockSpec((tile, PW), lambda i: (i, 0))] * 4,
        out_specs=pl.BlockSpec((tile, HP), lambda i: (i, 0)),
        out_shape=jax.ShapeDtypeStruct((NA, HP), _f32),
    )(sa, sb, sc0, sc1)


def _msg_kernel(ha_ref, hb_ref, hc_ref, g_ref, wm_ref, b_ref,
                oa_ref, ob_ref, oc_ref):
    m = g_ref[...] - jnp.concatenate(
        [ha_ref[...], hb_ref[...], hc_ref[...]], axis=1)
    y = jnp.maximum(_dotT(m, wm_ref[...]) + b_ref[...], 0.0)
    _split3(y, (oa_ref, ob_ref, oc_ref))


def _msg_pass(ha, hb, hc, g, wm, wmb, tile):
    pc = jax.ShapeDtypeStruct((NB, PW), _f32)
    return pl.pallas_call(
        _msg_kernel,
        grid=(NB // tile,),
        in_specs=[pl.BlockSpec((tile, PW), lambda i: (i, 0))] * 3
                 + [pl.BlockSpec((tile, HP), lambda i: (i, 0))]
                 + [pl.BlockSpec((HP, HP), lambda i: (0, 0)),
                    pl.BlockSpec((1, HP), lambda i: (0, 0))],
        out_specs=[pl.BlockSpec((tile, PW), lambda i: (i, 0))] * 3,
        out_shape=[pc, pc, pc],
    )(ha, hb, hc, g, wm, wmb)


def _msg_edge_kernel(ha_ref, hb_ref, hc_ref, g_ref, wm_ref, b_ref,
                     ew_ref, eb_ref, oa_ref, ob_ref, oc_ref, e_ref):
    m = g_ref[...] - jnp.concatenate(
        [ha_ref[...], hb_ref[...], hc_ref[...]], axis=1)
    y = jnp.maximum(_dotT(m, wm_ref[...]) + b_ref[...], 0.0)
    _split3(y, (oa_ref, ob_ref, oc_ref))
    e_ref[...] = _dotT(y, ew_ref[...]) + eb_ref[...]


def _msg_edge_pass(ha, hb, hc, g, wm, wmb, ew, eb, tile):
    pc = jax.ShapeDtypeStruct((NB, PW), _f32)
    return pl.pallas_call(
        _msg_edge_kernel,
        grid=(NB // tile,),
        in_specs=[pl.BlockSpec((tile, PW), lambda i: (i, 0))] * 3
                 + [pl.BlockSpec((tile, HP), lambda i: (i, 0))]
                 + [pl.BlockSpec((HP, HP), lambda i: (0, 0)),
                    pl.BlockSpec((1, HP), lambda i: (0, 0)),
                    pl.BlockSpec((EF, HP), lambda i: (0, 0)),
                    pl.BlockSpec((1, EF), lambda i: (0, 0))],
        out_specs=[pl.BlockSpec((tile, PW), lambda i: (i, 0))] * 3
                  + [pl.BlockSpec((tile, EF), lambda i: (i, 0))],
        out_shape=[pc, pc, pc, jax.ShapeDtypeStruct((NB, EF), _f32)],
    )(ha, hb, hc, g, wm, wmb, ew, eb)


_ATILE = 400
_MTILE = _ATILE // MOL


def _atoms_kernel(fa_ref, sa_ref, sb_ref, sc0_ref, sc1_ref, wa_ref, wab_ref,
                  nw_ref, nb_ref, np_ref, mol_ref):
    s3 = jnp.concatenate(
        [sa_ref[...], sb_ref[...], sc0_ref[...] + sc1_ref[...]], axis=1)
    ah = jnp.maximum(_dotT(fa_ref[...], wa_ref[...]) + wab_ref[...] + s3, 0.0)
    np_ref[...] = _dotT(ah, nw_ref[...]) + nb_ref[...]
    m_ids = lax.broadcasted_iota(jnp.int32, (_MTILE, _ATILE), 0)
    a_ids = lax.broadcasted_iota(jnp.int32, (_MTILE, _ATILE), 1) // MOL
    pool = (m_ids == a_ids).astype(_f32)
    mol_ref[...] = lax.dot_general(pool, ah, (((1,), (0,)), ((), ())),
                                   preferred_element_type=_f32,
                                   precision=_HI)


def _atoms_pass(f_atoms, sa, sb, sc0, sc1, wa, wab, nw, nb):
    return pl.pallas_call(
        _atoms_kernel,
        grid=(NA // _ATILE,),
        in_specs=[pl.BlockSpec((_ATILE, AF), lambda i: (i, 0))]
                 + [pl.BlockSpec((_ATILE, PW), lambda i: (i, 0))] * 4
                 + [pl.BlockSpec((HP, AF), lambda i: (0, 0)),
                    pl.BlockSpec((1, HP), lambda i: (0, 0)),
                    pl.BlockSpec((AF, HP), lambda i: (0, 0)),
                    pl.BlockSpec((1, AF), lambda i: (0, 0))],
        out_specs=[pl.BlockSpec((_ATILE, AF), lambda i: (i, 0)),
                   pl.BlockSpec((_MTILE, HP), lambda i: (i, 0))],
        out_shape=[jax.ShapeDtypeStruct((NA, AF), _f32),
                   jax.ShapeDtypeStruct((NM, HP), _f32)],
    )(f_atoms, sa, sb, sc0, sc1, wa, wab, nw, nb)


def _graph_kernel(mol_ref, g1_ref, g1b_ref, g2_ref, g2b_ref, o_ref):
    gh = jnp.maximum(_dotT(mol_ref[...], g1_ref[...]) + g1b_ref[...], 0.0)
    o_ref[...] = lax.dot_general(g2_ref[...], gh, (((1,), (1,)), ((), ())),
                                 preferred_element_type=_f32,
                                 precision=_PREC) + g2b_ref[...]


def _graph_pass(mol, g1, g1b, g2, g2b):
    return pl.pallas_call(
        _graph_kernel,
        grid=(1,),
        in_specs=[pl.BlockSpec((NM, HP), lambda i: (0, 0)),
                  pl.BlockSpec((HP, HP), lambda i: (0, 0)),
                  pl.BlockSpec((1, HP), lambda i: (0, 0)),
                  pl.BlockSpec((1, HP), lambda i: (0, 0)),
                  pl.BlockSpec((1, 1), lambda i: (0, 0))],
        out_specs=pl.BlockSpec((1, NM), lambda i: (0, 0)),
        out_shape=jax.ShapeDtypeStruct((1, NM), _f32),
    )(mol, g1, g1b, g2, g2b)


def _pad2(a, rows, cols):
    return jnp.pad(a, ((0, rows - a.shape[0]), (0, cols - a.shape[1])))


def kernel(f_atoms, f_bonds, b2a, a_scope, Wi_w, Wi_b, Wm_w, Wm_b, Wa_w, Wa_b,
           node_w, node_b, edge_w, edge_b, g1_w, g1_b, g2_w, g2_b):
    b2a = b2a.astype(jnp.int32)
    even = (lax.iota(jnp.int32, NB) % 2) == 0
    b2a_rev = jnp.where(even, jnp.roll(b2a, -1), jnp.roll(b2a, 1))

    wi = _pad2(Wi_w, HP, BF)
    wib = _pad2(Wi_b[None, :], 1, HP)
    wm = _pad2(Wm_w, HP, HP)
    wmb = _pad2(Wm_b[None, :], 1, HP)
    wa = _pad2(Wa_w, HP, AF)
    wab = _pad2(Wa_b[None, :], 1, HP)
    nw = _pad2(node_w, AF, HP)
    nb = node_b[None, :]
    ew = _pad2(edge_w, EF, HP)
    eb = edge_b[None, :]
    g1 = _pad2(g1_w, HP, HP)
    g1b = _pad2(g1_b[None, :], 1, HP)
    g2 = _pad2(g2_w, 1, HP)
    g2b = g2_b[None, :]
    zrows = jnp.zeros((_RPS, PW), _f32)

    ha, hb, hc = _h0_pass(f_bonds, wi, wib, tile=512)

    idx_pairs = ((b2a, b2a_rev), (b2a_rev, b2a), (b2a, b2a_rev))
    edge_rev = None
    for d in range(3):
        sc_idx, g_idx = idx_pairs[d]
        sa, sb, sc0, sc1 = _sc_scatter3(ha, hb, hc, sc_idx, zrows)
        s = _s_pass(sa, sb, sc0, sc1, tile=2000)
        g = _sc_gather(s, g_idx)
        if d < 2:
            ha, hb, hc = _msg_pass(ha, hb, hc, g, wm, wmb, tile=512)
        else:
            ha, hb, hc, edge_rev = _msg_edge_pass(ha, hb, hc, g, wm, wmb,
                                                  ew, eb, tile=512)

    sa, sb, sc0, sc1 = _sc_scatter3(ha, hb, hc, b2a_rev, zrows)
    node_pred, mol = _atoms_pass(f_atoms, sa, sb, sc0, sc1, wa, wab, nw, nb)
    graph = _graph_pass(mol, g1, g1b, g2, g2b)

    edge_pred = jnp.where(even[:, None], jnp.roll(edge_rev, -1, axis=0),
                          jnp.roll(edge_rev, 1, axis=0))
    return node_pred, edge_pred, graph[0]


# speedup vs baseline: 2.2552x; 1.1631x over previous
import functools

import jax
import jax.numpy as jnp
from jax import lax
from jax.experimental import pallas as pl
from jax.experimental.pallas import tpu as pltpu
from jax.experimental.pallas import tpu_sc as plsc

NB = 320000
NA = 10000
PW = 128
NP = 3
HP = PW * NP
NM = 400
MOL = 25
AF = 128
BF = 144
EF = 16

_NC, _NS = 2, 16
_NW = _NC * _NS
_SCH = 80
NAP = 10240
_RPS = NAP // _NS

_PREC = jax.lax.Precision.DEFAULT
_HI = jax.lax.Precision.HIGHEST
_f32 = jnp.float32


def _sc_scatter3(ha, hb, hc, idx2, zrows):
    mesh = plsc.VectorSubcoreMesh(core_axis_name="c", subcore_axis_name="s")
    out = jax.ShapeDtypeStruct((NAP, PW), _f32)

    @functools.partial(
        pl.kernel,
        out_type=[out, out, out, out],
        mesh=mesh,
        scratch_types=[
            pltpu.VMEM_SHARED((NAP, PW), _f32),
            pltpu.VMEM((_SCH,), jnp.int32),
            pltpu.VMEM((_SCH,), jnp.int32),
            pltpu.VMEM((_SCH, PW), _f32),
            pltpu.VMEM((_SCH, PW), _f32),
            pltpu.SemaphoreType.DMA,
            pltpu.SemaphoreType.DMA,
        ],
    )
    def k(a_hbm, b_hbm, c_hbm, idx_hbm, z_hbm, sa, sb, sc0, sc1,
          acc, idxa, idxb, bufa, bufb, sema, semb):
        c = lax.axis_index("c")
        s = lax.axis_index("s")
        row0 = s * _RPS
        npw = NB // _NW // _SCH

        def start(x_hbm, bond0, j, buf, ib, sem):
            pltpu.async_copy(x_hbm.at[pl.ds(bond0 + j * _SCH, _SCH)], buf, sem)
            pltpu.async_copy(idx_hbm.at[pl.ds(bond0 + j * _SCH, _SCH)], ib, sem)

        def wait(x_hbm, bond0, buf, ib, sem):
            pltpu.make_async_copy(x_hbm.at[pl.ds(bond0, _SCH)], buf, sem).wait()
            pltpu.make_async_copy(idx_hbm.at[pl.ds(bond0, _SCH)], ib, sem).wait()

        def scan(x_hbm, w):
            bond0 = w * (NB // _NW)
            start(x_hbm, bond0, 0, bufa, idxa, sema)

            @pl.loop(0, npw - 1, step=2)
            def _(j):
                start(x_hbm, bond0, j + 1, bufb, idxb, semb)
                wait(x_hbm, bond0, bufa, idxa, sema)
                pltpu.sync_copy(bufa, acc.at[idxa], add=True)
                start(x_hbm, bond0, j + 2, bufa, idxa, sema)
                wait(x_hbm, bond0, bufb, idxb, semb)
                pltpu.sync_copy(bufb, acc.at[idxb], add=True)

            wait(x_hbm, bond0, bufa, idxa, sema)
            pltpu.sync_copy(bufa, acc.at[idxa], add=True)

        pltpu.sync_copy(z_hbm, acc.at[pl.ds(row0, _RPS)])
        plsc.subcore_barrier()

        @pl.when(c == 0)
        def _():
            scan(a_hbm, 2 * s)
            scan(a_hbm, 2 * s + 1)

        @pl.when(c == 1)
        def _():
            scan(b_hbm, 2 * s)
            scan(b_hbm, 2 * s + 1)

        plsc.subcore_barrier()

        @pl.when(c == 0)
        def _():
            pltpu.sync_copy(acc.at[pl.ds(row0, _RPS)], sa.at[pl.ds(row0, _RPS)])

        @pl.when(c == 1)
        def _():
            pltpu.sync_copy(acc.at[pl.ds(row0, _RPS)], sb.at[pl.ds(row0, _RPS)])

        pltpu.sync_copy(z_hbm, acc.at[pl.ds(row0, _RPS)])
        plsc.subcore_barrier()
        scan(c_hbm, c * _NS + s)
        plsc.subcore_barrier()

        @pl.when(c == 0)
        def _():
            pltpu.sync_copy(acc.at[pl.ds(row0, _RPS)], sc0.at[pl.ds(row0, _RPS)])

        @pl.when(c == 1)
        def _():
            pltpu.sync_copy(acc.at[pl.ds(row0, _RPS)], sc1.at[pl.ds(row0, _RPS)])

    return k(ha, hb, hc, idx2, zrows)


def _sc_gather(t, idx):
    mesh = plsc.VectorSubcoreMesh(core_axis_name="c", subcore_axis_name="s")
    npw = NB // _NW // _SCH

    @functools.partial(
        pl.kernel,
        out_type=jax.ShapeDtypeStruct((NB, HP), _f32),
        mesh=mesh,
        scratch_types=[
            pltpu.VMEM((NB // _NW,), jnp.int32),
            pltpu.VMEM((_SCH, HP), _f32),
            pltpu.VMEM((_SCH, HP), _f32),
            pltpu.SemaphoreType.DMA,
            pltpu.SemaphoreType.DMA,
        ],
    )
    def k(t_hbm, idx_hbm, g_hbm, idxb, bufa, bufb, sema, semb):
        wid = lax.axis_index("c") * _NS + lax.axis_index("s")
        base = wid * (NB // _NW)
        pltpu.sync_copy(idx_hbm.at[pl.ds(base, NB // _NW)], idxb)

        def gstart(j, buf, sem):
            pltpu.async_copy(t_hbm.at[idxb.at[pl.ds(j * _SCH, _SCH)]], buf, sem)

        def gwait(buf, sem):
            pltpu.make_async_copy(t_hbm.at[idxb.at[pl.ds(0, _SCH)]], buf,
                                  sem).wait()

        def wout(j, buf):
            pltpu.sync_copy(buf, g_hbm.at[pl.ds(base + j * _SCH, _SCH)])

        gstart(0, bufa, sema)

        @pl.loop(0, npw - 1, step=2)
        def _(j):
            gstart(j + 1, bufb, semb)
            gwait(bufa, sema)
            wout(j, bufa)
            gstart(j + 2, bufa, sema)
            gwait(bufb, semb)
            wout(j + 1, bufb)

        gwait(bufa, sema)
        wout(npw - 1, bufa)

    return k(t, idx)


def _dotT(x, w, prec=_PREC):
    return lax.dot_general(x, w, (((1,), (1,)), ((), ())),
                           preferred_element_type=_f32, precision=prec)


def _split3(y, refs):
    for p, r in enumerate(refs):
        r[...] = y[:, p * PW:(p + 1) * PW]


def _h0_kernel(x_ref, w_ref, b_ref, oa_ref, ob_ref, oc_ref):
    y = jnp.maximum(_dotT(x_ref[...], w_ref[...]) + b_ref[...], 0.0)
    _split3(y, (oa_ref, ob_ref, oc_ref))


def _h0_pass(f_bonds, wi, wib, tile):
    pc = jax.ShapeDtypeStruct((NB, PW), _f32)
    return pl.pallas_call(
        _h0_kernel,
        grid=(NB // tile,),
        in_specs=[pl.BlockSpec((tile, BF), lambda i: (i, 0)),
                  pl.BlockSpec((HP, BF), lambda i: (0, 0)),
                  pl.BlockSpec((1, HP), lambda i: (0, 0))],
        out_specs=[pl.BlockSpec((tile, PW), lambda i: (i, 0))] * 3,
        out_shape=[pc, pc, pc],
    )(f_bonds, wi, wib)


def _s_kernel(sa_ref, sb_ref, sc0_ref, sc1_ref, s_ref):
    s_ref[...] = jnp.concatenate(
        [sa_ref[...], sb_ref[...], sc0_ref[...] + sc1_ref[...]], axis=1)


def _s_pass(sa, sb, sc0, sc1, tile):
    return pl.pallas_call(
        _s_kernel,
        grid=(NA // tile,),
        in_specs=[pl.BlockSpec((tile, PW), lambda i: (i, 0))] * 4,
        out_specs=pl.BlockSpec((tile, HP), lambda i: (i, 0)),
        out_shape=jax.ShapeDtypeStruct((NA, HP), _f32),
    )(sa, sb, sc0, sc1)


def _msg_kernel(ha_ref, hb_ref, hc_ref, g_ref, wm_ref, b_ref,
                oa_ref, ob_ref, oc_ref):
    m = g_ref[...] - jnp.concatenate(
        [ha_ref[...], hb_ref[...], hc_ref[...]], axis=1)
    y = jnp.maximum(_dotT(m, wm_ref[...]) + b_ref[...], 0.0)
    _split3(y, (oa_ref, ob_ref, oc_ref))


def _msg_pass(ha, hb, hc, g, wm, wmb, tile):
    pc = jax.ShapeDtypeStruct((NB, PW), _f32)
    return pl.pallas_call(
        _msg_kernel,
        grid=(NB // tile,),
        in_specs=[pl.BlockSpec((tile, PW), lambda i: (i, 0))] * 3
                 + [pl.BlockSpec((tile, HP), lambda i: (i, 0))]
                 + [pl.BlockSpec((HP, HP), lambda i: (0, 0)),
                    pl.BlockSpec((1, HP), lambda i: (0, 0))],
        out_specs=[pl.BlockSpec((tile, PW), lambda i: (i, 0))] * 3,
        out_shape=[pc, pc, pc],
    )(ha, hb, hc, g, wm, wmb)


def _msg_edge_kernel(ha_ref, hb_ref, hc_ref, g_ref, wm_ref, b_ref,
                     ew_ref, eb_ref, oa_ref, ob_ref, oc_ref, e_ref):
    m = g_ref[...] - jnp.concatenate(
        [ha_ref[...], hb_ref[...], hc_ref[...]], axis=1)
    y = jnp.maximum(_dotT(m, wm_ref[...]) + b_ref[...], 0.0)
    _split3(y, (oa_ref, ob_ref, oc_ref))
    e_ref[...] = _dotT(y, ew_ref[...]) + eb_ref[...]


def _msg_edge_pass(ha, hb, hc, g, wm, wmb, ew, eb, tile):
    pc = jax.ShapeDtypeStruct((NB, PW), _f32)
    return pl.pallas_call(
        _msg_edge_kernel,
        grid=(NB // tile,),
        in_specs=[pl.BlockSpec((tile, PW), lambda i: (i, 0))] * 3
                 + [pl.BlockSpec((tile, HP), lambda i: (i, 0))]
                 + [pl.BlockSpec((HP, HP), lambda i: (0, 0)),
                    pl.BlockSpec((1, HP), lambda i: (0, 0)),
                    pl.BlockSpec((EF, HP), lambda i: (0, 0)),
                    pl.BlockSpec((1, EF), lambda i: (0, 0))],
        out_specs=[pl.BlockSpec((tile, PW), lambda i: (i, 0))] * 3
                  + [pl.BlockSpec((tile, EF), lambda i: (i, 0))],
        out_shape=[pc, pc, pc, jax.ShapeDtypeStruct((NB, EF), _f32)],
    )(ha, hb, hc, g, wm, wmb, ew, eb)


_ATILE = 400
_MTILE = _ATILE // MOL


def _atoms_kernel(fa_ref, sa_ref, sb_ref, sc0_ref, sc1_ref, wa_ref, wab_ref,
                  nw_ref, nb_ref, np_ref, mol_ref):
    s3 = jnp.concatenate(
        [sa_ref[...], sb_ref[...], sc0_ref[...] + sc1_ref[...]], axis=1)
    ah = jnp.maximum(_dotT(fa_ref[...], wa_ref[...]) + wab_ref[...] + s3, 0.0)
    np_ref[...] = _dotT(ah, nw_ref[...]) + nb_ref[...]
    m_ids = lax.broadcasted_iota(jnp.int32, (_MTILE, _ATILE), 0)
    a_ids = lax.broadcasted_iota(jnp.int32, (_MTILE, _ATILE), 1) // MOL
    pool = (m_ids == a_ids).astype(_f32)
    mol_ref[...] = lax.dot_general(pool, ah, (((1,), (0,)), ((), ())),
                                   preferred_element_type=_f32,
                                   precision=_HI)


def _atoms_pass(f_atoms, sa, sb, sc0, sc1, wa, wab, nw, nb):
    return pl.pallas_call(
        _atoms_kernel,
        grid=(NA // _ATILE,),
        in_specs=[pl.BlockSpec((_ATILE, AF), lambda i: (i, 0))]
                 + [pl.BlockSpec((_ATILE, PW), lambda i: (i, 0))] * 4
                 + [pl.BlockSpec((HP, AF), lambda i: (0, 0)),
                    pl.BlockSpec((1, HP), lambda i: (0, 0)),
                    pl.BlockSpec((AF, HP), lambda i: (0, 0)),
                    pl.BlockSpec((1, AF), lambda i: (0, 0))],
        out_specs=[pl.BlockSpec((_ATILE, AF), lambda i: (i, 0)),
                   pl.BlockSpec((_MTILE, HP), lambda i: (i, 0))],
        out_shape=[jax.ShapeDtypeStruct((NA, AF), _f32),
                   jax.ShapeDtypeStruct((NM, HP), _f32)],
    )(f_atoms, sa, sb, sc0, sc1, wa, wab, nw, nb)


def _graph_kernel(mol_ref, g1_ref, g1b_ref, g2_ref, g2b_ref, o_ref):
    gh = jnp.maximum(_dotT(mol_ref[...], g1_ref[...]) + g1b_ref[...], 0.0)
    o_ref[...] = lax.dot_general(g2_ref[...], gh, (((1,), (1,)), ((), ())),
                                 preferred_element_type=_f32,
                                 precision=_PREC) + g2b_ref[...]


def _graph_pass(mol, g1, g1b, g2, g2b):
    return pl.pallas_call(
        _graph_kernel,
        grid=(1,),
        in_specs=[pl.BlockSpec((NM, HP), lambda i: (0, 0)),
                  pl.BlockSpec((HP, HP), lambda i: (0, 0)),
                  pl.BlockSpec((1, HP), lambda i: (0, 0)),
                  pl.BlockSpec((1, HP), lambda i: (0, 0)),
                  pl.BlockSpec((1, 1), lambda i: (0, 0))],
        out_specs=pl.BlockSpec((1, NM), lambda i: (0, 0)),
        out_shape=jax.ShapeDtypeStruct((1, NM), _f32),
    )(mol, g1, g1b, g2, g2b)


def _pad2(a, rows, cols):
    return jnp.pad(a, ((0, rows - a.shape[0]), (0, cols - a.shape[1])))


def kernel(f_atoms, f_bonds, b2a, a_scope, Wi_w, Wi_b, Wm_w, Wm_b, Wa_w, Wa_b,
           node_w, node_b, edge_w, edge_b, g1_w, g1_b, g2_w, g2_b):
    b2a = b2a.astype(jnp.int32)
    even = (lax.iota(jnp.int32, NB) % 2) == 0
    b2a_rev = jnp.where(even, jnp.roll(b2a, -1), jnp.roll(b2a, 1))

    wi = _pad2(Wi_w, HP, BF)
    wib = _pad2(Wi_b[None, :], 1, HP)
    wm = _pad2(Wm_w, HP, HP)
    wmb = _pad2(Wm_b[None, :], 1, HP)
    wa = _pad2(Wa_w, HP, AF)
    wab = _pad2(Wa_b[None, :], 1, HP)
    nw = _pad2(node_w, AF, HP)
    nb = node_b[None, :]
    ew = _pad2(edge_w, EF, HP)
    eb = edge_b[None, :]
    g1 = _pad2(g1_w, HP, HP)
    g1b = _pad2(g1_b[None, :], 1, HP)
    g2 = _pad2(g2_w, 1, HP)
    g2b = g2_b[None, :]
    zrows = jnp.zeros((_RPS, PW), _f32)

    ha, hb, hc = _h0_pass(f_bonds, wi, wib, tile=1024)

    idx_pairs = ((b2a, b2a_rev), (b2a_rev, b2a), (b2a, b2a_rev))
    edge_rev = None
    for d in range(3):
        sc_idx, g_idx = idx_pairs[d]
        sa, sb, sc0, sc1 = _sc_scatter3(ha, hb, hc, sc_idx, zrows)
        s = _s_pass(sa, sb, sc0, sc1, tile=2000)
        g = _sc_gather(s, g_idx)
        if d < 2:
            ha, hb, hc = _msg_pass(ha, hb, hc, g, wm, wmb, tile=1024)
        else:
            ha, hb, hc, edge_rev = _msg_edge_pass(ha, hb, hc, g, wm, wmb,
                                                  ew, eb, tile=1024)

    sa, sb, sc0, sc1 = _sc_scatter3(ha, hb, hc, b2a_rev, zrows)
    node_pred, mol = _atoms_pass(f_atoms, sa, sb, sc0, sc1, wa, wab, nw, nb)
    graph = _graph_pass(mol, g1, g1b, g2, g2b)

    edge_pred = jnp.where(even[:, None], jnp.roll(edge_rev, -1, axis=0),
                          jnp.roll(edge_rev, 1, axis=0))
    return node_pred, edge_pred, graph[0]
